# baseline (speedup 1.0000x reference)
import jax
import jax.numpy as jnp
from jax import lax
from jax.experimental import pallas as pl
from jax.experimental.pallas import tpu as pltpu
from jax.experimental.pallas import tpu_sc as plsc

N = 10000
F = 128
E = 320000
NC, NS = 2, 16
NW = NC * NS
CHUNK = 128
CPW = 80
EPAD = NW * CPW * CHUNK
NPAD = 10240
SLAB = NPAD // NS
TPAD = 10016

_MESH = plsc.VectorSubcoreMesh(
    core_axis_name="c", subcore_axis_name="s", num_cores=NC, num_subcores=NS)


def _make_scat_body(gather):
    def body_fn(tab_h, src_h, dst_h, zrow_h, out_h,
                sidx_v, didx_v, rows_v, acc):
        cid = lax.axis_index("c")
        sid = lax.axis_index("s")
        wid = sid * NC + cid
        base = sid * SLAB
        pltpu.sync_copy(zrow_h, rows_v)
        for jz in range(SLAB // CHUNK):
            pltpu.sync_copy(rows_v, acc.at[pl.ds(base + jz * CHUNK, CHUNK)])
        plsc.subcore_barrier()
        if not gather:
            pltpu.sync_copy(tab_h.at[pl.ds(0, CHUNK)], rows_v)

        def body(j, carry):
            pltpu.sync_copy(dst_h.at[wid, j], didx_v)
            if gather:
                pltpu.sync_copy(src_h.at[wid, j], sidx_v)
                pltpu.sync_copy(tab_h.at[sidx_v], rows_v)
            pltpu.sync_copy(rows_v, acc.at[didx_v], add=True)
            return carry

        lax.fori_loop(0, CPW, body, 0)
        plsc.subcore_barrier()
        for jz in range(SLAB // CHUNK):
            pltpu.sync_copy(acc.at[pl.ds(base + jz * CHUNK, CHUNK)], rows_v)
            pltpu.sync_copy(rows_v,
                            out_h.at[cid, pl.ds(base + jz * CHUNK, CHUNK)])

    return body_fn


def _scat_pipe_body(tab_h, src_h, dst_h, zrow_h, out_h,
                    sidx_a, didx_a, sidx_b, didx_b, rows_a, rows_b,
                    sem_a, sem_b, acc):
    cid = lax.axis_index("c")
    sid = lax.axis_index("s")
    wid = sid * NC + cid
    base = sid * SLAB
    pltpu.sync_copy(zrow_h, rows_a)
    for jz in range(SLAB // CHUNK):
        pltpu.sync_copy(rows_a, acc.at[pl.ds(base + jz * CHUNK, CHUNK)])
    plsc.subcore_barrier()
    pltpu.sync_copy(src_h.at[wid, 0], sidx_a)
    pltpu.sync_copy(dst_h.at[wid, 0], didx_a)
    pltpu.async_copy(tab_h.at[sidx_a], rows_a, sem_a)

    def body(g2, carry):
        g = 2 * g2
        pltpu.sync_copy(src_h.at[wid, g + 1], sidx_b)
        pltpu.sync_copy(dst_h.at[wid, g + 1], didx_b)
        pltpu.async_copy(tab_h.at[sidx_b], rows_b, sem_b)
        pltpu.make_async_copy(tab_h.at[sidx_a], rows_a, sem_a).wait()
        pltpu.sync_copy(rows_a, acc.at[didx_a], add=True)
        gnext = jnp.minimum(g + 2, CPW - 1)
        pltpu.sync_copy(src_h.at[wid, gnext], sidx_a)
        pltpu.sync_copy(dst_h.at[wid, gnext], didx_a)
        pltpu.async_copy(tab_h.at[sidx_a], rows_a, sem_a)
        pltpu.make_async_copy(tab_h.at[sidx_b], rows_b, sem_b).wait()
        pltpu.sync_copy(rows_b, acc.at[didx_b], add=True)
        return carry

    lax.fori_loop(0, CPW // 2, body, 0)
    pltpu.make_async_copy(tab_h.at[sidx_a], rows_a, sem_a).wait()
    plsc.subcore_barrier()
    for jz in range(SLAB // CHUNK):
        pltpu.sync_copy(acc.at[pl.ds(base + jz * CHUNK, CHUNK)], rows_a)
        pltpu.sync_copy(rows_a,
                        out_h.at[cid, pl.ds(base + jz * CHUNK, CHUNK)])


_scat_kernel = pl.kernel(
    _scat_pipe_body,
    out_type=jax.ShapeDtypeStruct((NC, NPAD, F), jnp.float32),
    mesh=_MESH,
    scratch_types=[
        pltpu.VMEM((CHUNK,), jnp.int32),
        pltpu.VMEM((CHUNK,), jnp.int32),
        pltpu.VMEM((CHUNK,), jnp.int32),
        pltpu.VMEM((CHUNK,), jnp.int32),
        pltpu.VMEM((CHUNK, F), jnp.float32),
        pltpu.VMEM((CHUNK, F), jnp.float32),
        pltpu.SemaphoreType.DMA,
        pltpu.SemaphoreType.DMA,
        pltpu.VMEM_SHARED((NPAD, F), jnp.float32),
    ],
)

_deg_kernel = pl.kernel(
    _make_scat_body(gather=False),
    out_type=jax.ShapeDtypeStruct((NC, NPAD, F), jnp.float32),
    mesh=_MESH,
    scratch_types=[
        pltpu.VMEM((CHUNK,), jnp.int32),
        pltpu.VMEM((CHUNK,), jnp.int32),
        pltpu.VMEM((CHUNK, F), jnp.float32),
        pltpu.VMEM_SHARED((NPAD, F), jnp.float32),
    ],
)


def _prep_body(degp_ref, x_ref, xs_ref, dinv_ref):
    deg = degp_ref[0][:, :1] + degp_ref[1][:, :1]
    dinv = lax.rsqrt(deg + 1.0)
    dinv_ref[...] = dinv
    xs_ref[:N, :] = x_ref[...] * dinv[:N]
    xs_ref[N:, :] = jnp.zeros((TPAD - N, F), jnp.float32)


_prep_kernel = pl.pallas_call(
    _prep_body,
    out_shape=(jax.ShapeDtypeStruct((TPAD, F), jnp.float32),
               jax.ShapeDtypeStruct((NPAD, 1), jnp.float32)),
)


def _mid_body(acc_ref, xs_ref, dinv_ref, w1_ref, b1_ref, g_ref, bt_ref,
              w2_ref, ys_ref):
    dinv = dinv_ref[...][:N]
    a = acc_ref[0][:N] + acc_ref[1][:N] + xs_ref[...][:N]
    p1 = a * dinv
    t = jnp.dot(p1, w1_ref[...], preferred_element_type=jnp.float32,
                precision=lax.Precision.HIGHEST) + b1_ref[...]
    mean = jnp.mean(t, axis=0, keepdims=True)
    c = t - mean
    var = jnp.mean(c * c, axis=0, keepdims=True)
    h = jnp.maximum(c * lax.rsqrt(var + 1e-5) * g_ref[...] + bt_ref[...], 0.0)
    q = jnp.dot(h, w2_ref[...], preferred_element_type=jnp.float32,
                precision=lax.Precision.HIGHEST)
    ys_ref[:N, :] = q * dinv
    ys_ref[N:, :] = jnp.zeros((TPAD - N, F), jnp.float32)


_mid_kernel = pl.pallas_call(
    _mid_body,
    out_shape=jax.ShapeDtypeStruct((TPAD, F), jnp.float32),
)


def _fin_body(acc_ref, ys_ref, dinv_ref, b2_ref, out_ref):
    a = acc_ref[0][:N] + acc_ref[1][:N] + ys_ref[...][:N]
    out_ref[...] = a * dinv_ref[...][:N] + b2_ref[...]


_fin_kernel = pl.pallas_call(
    _fin_body,
    out_shape=jax.ShapeDtypeStruct((N, F), jnp.float32),
)


def kernel(x, edge_index, W1, b1, gamma, beta, W2, b2):
    ei = edge_index.astype(jnp.int32)
    pad = jnp.full((EPAD - E,), N, jnp.int32)
    srcp = jnp.concatenate([ei[0], pad]).reshape(NW, CPW, CHUNK)
    dstp = jnp.concatenate([ei[1], pad]).reshape(NW, CPW, CHUNK)
    ones_row = jnp.ones((CHUNK, F), jnp.float32)
    zeros_row = jnp.zeros((CHUNK, F), jnp.float32)

    degp = _deg_kernel(ones_row, srcp, dstp, zeros_row)
    xs, dinv = _prep_kernel(degp, x)
    acc1 = _scat_kernel(xs, srcp, dstp, zeros_row)
    ys = _mid_kernel(acc1, xs, dinv, W1, b1.reshape(1, -1),
                     gamma.reshape(1, -1), beta.reshape(1, -1), W2)
    acc2 = _scat_kernel(ys, srcp, dstp, zeros_row)
    return _fin_kernel(acc2, ys, dinv, b2.reshape(1, -1))


# speedup vs baseline: 1.1485x; 1.1485x over previous
import jax
import jax.numpy as jnp
from jax import lax
from jax.experimental import pallas as pl
from jax.experimental.pallas import tpu as pltpu
from jax.experimental.pallas import tpu_sc as plsc

N = 10000
F = 128
E = 320000
NC, NS = 2, 16
NW = NC * NS
CHUNK = 128
CPW = 80
CPT = 2 * CPW
CP_FAST, CP_SLOW = 120, 40
FAST_CID = 1
EPAD = NW * CPW * CHUNK
NPAD = 10240
SLAB = NPAD // NS
TPAD = 10016

_MESH = plsc.VectorSubcoreMesh(
    core_axis_name="c", subcore_axis_name="s", num_cores=NC, num_subcores=NS)


def _deg_body(ones_h, src_h, dst_h, zrow_h, out_h, didx_v, rows_v, acc):
    cid = lax.axis_index("c")
    sid = lax.axis_index("s")
    base = sid * SLAB
    coff = cid * CPW
    pltpu.sync_copy(zrow_h, rows_v)
    for jz in range(SLAB // CHUNK):
        pltpu.sync_copy(rows_v, acc.at[pl.ds(base + jz * CHUNK, CHUNK)])
    plsc.subcore_barrier()
    pltpu.sync_copy(ones_h.at[pl.ds(0, CHUNK)], rows_v)

    def body(j, carry):
        pltpu.sync_copy(dst_h.at[sid, coff + j], didx_v)
        pltpu.sync_copy(rows_v, acc.at[didx_v], add=True)
        return carry

    lax.fori_loop(0, CPW, body, 0)
    plsc.subcore_barrier()
    for jz in range(SLAB // CHUNK):
        pltpu.sync_copy(acc.at[pl.ds(base + jz * CHUNK, CHUNK)], rows_v)
        pltpu.sync_copy(rows_v,
                        out_h.at[cid, pl.ds(base + jz * CHUNK, CHUNK)])


def _scat_pipe_body(tab_h, src_h, dst_h, zrow_h, out_h,
                    sidx_a, didx_a, sidx_b, didx_b, rows_a, rows_b,
                    sem_a, sem_b, acc):
    cid = lax.axis_index("c")
    sid = lax.axis_index("s")
    base = sid * SLAB
    fast = cid == FAST_CID
    coff = jnp.where(fast, 0, CP_FAST)
    cnt = jnp.where(fast, CP_FAST, CP_SLOW)
    pltpu.sync_copy(zrow_h, rows_a)
    for jz in range(SLAB // CHUNK):
        pltpu.sync_copy(rows_a, acc.at[pl.ds(base + jz * CHUNK, CHUNK)])
    plsc.subcore_barrier()
    pltpu.sync_copy(src_h.at[sid, coff], sidx_a)
    pltpu.sync_copy(dst_h.at[sid, coff], didx_a)
    pltpu.async_copy(tab_h.at[sidx_a], rows_a, sem_a)

    def body(g2, carry):
        g = coff + 2 * g2
        pltpu.sync_copy(src_h.at[sid, g + 1], sidx_b)
        pltpu.sync_copy(dst_h.at[sid, g + 1], didx_b)
        pltpu.async_copy(tab_h.at[sidx_b], rows_b, sem_b)
        pltpu.make_async_copy(tab_h.at[sidx_a], rows_a, sem_a).wait()
        pltpu.sync_copy(rows_a, acc.at[didx_a], add=True)
        gnext = jnp.minimum(g + 2, coff + cnt - 1)
        pltpu.sync_copy(src_h.at[sid, gnext], sidx_a)
        pltpu.sync_copy(dst_h.at[sid, gnext], didx_a)
        pltpu.async_copy(tab_h.at[sidx_a], rows_a, sem_a)
        pltpu.make_async_copy(tab_h.at[sidx_b], rows_b, sem_b).wait()
        pltpu.sync_copy(rows_b, acc.at[didx_b], add=True)
        return carry

    lax.fori_loop(0, cnt // 2, body, 0)
    pltpu.make_async_copy(tab_h.at[sidx_a], rows_a, sem_a).wait()
    plsc.subcore_barrier()
    for jz in range(SLAB // CHUNK):
        pltpu.sync_copy(acc.at[pl.ds(base + jz * CHUNK, CHUNK)], rows_a)
        pltpu.sync_copy(rows_a,
                        out_h.at[cid, pl.ds(base + jz * CHUNK, CHUNK)])


_scat_kernel = pl.kernel(
    _scat_pipe_body,
    out_type=jax.ShapeDtypeStruct((NC, NPAD, F), jnp.float32),
    mesh=_MESH,
    scratch_types=[
        pltpu.VMEM((CHUNK,), jnp.int32),
        pltpu.VMEM((CHUNK,), jnp.int32),
        pltpu.VMEM((CHUNK,), jnp.int32),
        pltpu.VMEM((CHUNK,), jnp.int32),
        pltpu.VMEM((CHUNK, F), jnp.float32),
        pltpu.VMEM((CHUNK, F), jnp.float32),
        pltpu.SemaphoreType.DMA,
        pltpu.SemaphoreType.DMA,
        pltpu.VMEM_SHARED((NPAD, F), jnp.float32),
    ],
)

_deg_kernel = pl.kernel(
    _deg_body,
    out_type=jax.ShapeDtypeStruct((NC, NPAD, F), jnp.float32),
    mesh=_MESH,
    scratch_types=[
        pltpu.VMEM((CHUNK,), jnp.int32),
        pltpu.VMEM((CHUNK, F), jnp.float32),
        pltpu.VMEM_SHARED((NPAD, F), jnp.float32),
    ],
)


def _prep_body(degp_ref, x_ref, xs_ref, dinv_ref):
    deg = degp_ref[0][:, :1] + degp_ref[1][:, :1]
    dinv = lax.rsqrt(deg + 1.0)
    dinv_ref[...] = dinv
    xs_ref[:N, :] = x_ref[...] * dinv[:N]
    xs_ref[N:, :] = jnp.zeros((TPAD - N, F), jnp.float32)


_prep_kernel = pl.pallas_call(
    _prep_body,
    out_shape=(jax.ShapeDtypeStruct((TPAD, F), jnp.float32),
               jax.ShapeDtypeStruct((NPAD, 1), jnp.float32)),
)


def _mid_body(acc_ref, xs_ref, dinv_ref, w1_ref, b1_ref, g_ref, bt_ref,
              w2_ref, ys_ref):
    dinv = dinv_ref[...][:N]
    a = acc_ref[0][:N] + acc_ref[1][:N] + xs_ref[...][:N]
    p1 = a * dinv
    t = jnp.dot(p1, w1_ref[...], preferred_element_type=jnp.float32,
                precision=lax.Precision.HIGHEST) + b1_ref[...]
    mean = jnp.mean(t, axis=0, keepdims=True)
    c = t - mean
    var = jnp.mean(c * c, axis=0, keepdims=True)
    h = jnp.maximum(c * lax.rsqrt(var + 1e-5) * g_ref[...] + bt_ref[...], 0.0)
    q = jnp.dot(h, w2_ref[...], preferred_element_type=jnp.float32,
                precision=lax.Precision.HIGHEST)
    ys_ref[:N, :] = q * dinv
    ys_ref[N:, :] = jnp.zeros((TPAD - N, F), jnp.float32)


_mid_kernel = pl.pallas_call(
    _mid_body,
    out_shape=jax.ShapeDtypeStruct((TPAD, F), jnp.float32),
)


def _fin_body(acc_ref, ys_ref, dinv_ref, b2_ref, out_ref):
    a = acc_ref[0][:N] + acc_ref[1][:N] + ys_ref[...][:N]
    out_ref[...] = a * dinv_ref[...][:N] + b2_ref[...]


_fin_kernel = pl.pallas_call(
    _fin_body,
    out_shape=jax.ShapeDtypeStruct((N, F), jnp.float32),
)


def kernel(x, edge_index, W1, b1, gamma, beta, W2, b2):
    ei = edge_index.astype(jnp.int32)
    pad = jnp.full((EPAD - E,), N, jnp.int32)
    srcp = jnp.concatenate([ei[0], pad]).reshape(NS, CPT, CHUNK)
    dstp = jnp.concatenate([ei[1], pad]).reshape(NS, CPT, CHUNK)
    ones_row = jnp.ones((CHUNK, F), jnp.float32)
    zeros_row = jnp.zeros((CHUNK, F), jnp.float32)

    degp = _deg_kernel(ones_row, srcp, dstp, zeros_row)
    xs, dinv = _prep_kernel(degp, x)
    acc1 = _scat_kernel(xs, srcp, dstp, zeros_row)
    ys = _mid_kernel(acc1, xs, dinv, W1, b1.reshape(1, -1),
                     gamma.reshape(1, -1), beta.reshape(1, -1), W2)
    acc2 = _scat_kernel(ys, srcp, dstp, zeros_row)
    return _fin_kernel(acc2, ys, dinv, b2.reshape(1, -1))


# speedup vs baseline: 1.4867x; 1.2944x over previous
import jax
import jax.numpy as jnp
from jax import lax
from jax.experimental import pallas as pl
from jax.experimental.pallas import tpu as pltpu
from jax.experimental.pallas import tpu_sc as plsc

N = 10000
F = 128
E = 320000
NC, NS = 2, 16
NW = NC * NS
CHUNK = 128
CPW = 80
CPT = 2 * CPW
CP_FAST, CP_SLOW = 80, 80
FAST_CID = 1
BT = jnp.bfloat16
_SC_PARAMS = pltpu.CompilerParams(use_tc_tiling_on_sc=False)
EPAD = NW * CPW * CHUNK
NPAD = 10240
SLAB = NPAD // NS
TPAD = 10016

_MESH = plsc.VectorSubcoreMesh(
    core_axis_name="c", subcore_axis_name="s", num_cores=NC, num_subcores=NS)


def _deg_body(ones_h, src_h, dst_h, zrow_h, out_h, didx_v, rows_v, acc):
    cid = lax.axis_index("c")
    sid = lax.axis_index("s")
    base = sid * SLAB
    coff = cid * CPW
    pltpu.sync_copy(zrow_h, rows_v)
    for jz in range(SLAB // CHUNK):
        pltpu.sync_copy(rows_v, acc.at[pl.ds(base + jz * CHUNK, CHUNK)])
    plsc.subcore_barrier()
    pltpu.sync_copy(ones_h.at[pl.ds(0, CHUNK)], rows_v)

    def body(j, carry):
        pltpu.sync_copy(dst_h.at[sid, coff + j], didx_v)
        pltpu.sync_copy(rows_v, acc.at[didx_v], add=True)
        return carry

    lax.fori_loop(0, CPW, body, 0)
    plsc.subcore_barrier()
    for jz in range(SLAB // CHUNK):
        pltpu.sync_copy(acc.at[pl.ds(base + jz * CHUNK, CHUNK)], rows_v)
        pltpu.sync_copy(rows_v,
                        out_h.at[cid, pl.ds(base + jz * CHUNK, CHUNK)])


def _scat_pipe_body(tab_h, src_h, dst_h, zrow_h, out_h,
                    sidx_a, didx_a, sidx_b, didx_b, rows_a, rows_b,
                    sem_a, sem_b, acc):
    cid = lax.axis_index("c")
    sid = lax.axis_index("s")
    base = sid * SLAB
    fast = cid == FAST_CID
    coff = jnp.where(fast, 0, CP_FAST)
    cnt = jnp.where(fast, CP_FAST, CP_SLOW)
    pltpu.sync_copy(zrow_h, rows_a)
    for jz in range(SLAB // CHUNK):
        pltpu.sync_copy(rows_a, acc.at[pl.ds(base + jz * CHUNK, CHUNK)])
    plsc.subcore_barrier()
    pltpu.sync_copy(src_h.at[sid, coff], sidx_a)
    pltpu.sync_copy(dst_h.at[sid, coff], didx_a)
    pltpu.async_copy(tab_h.at[sidx_a], rows_a, sem_a)

    def body(g2, carry):
        g = coff + 2 * g2
        pltpu.sync_copy(src_h.at[sid, g + 1], sidx_b)
        pltpu.sync_copy(dst_h.at[sid, g + 1], didx_b)
        pltpu.async_copy(tab_h.at[sidx_b], rows_b, sem_b)
        pltpu.make_async_copy(tab_h.at[sidx_a], rows_a, sem_a).wait()
        pltpu.sync_copy(rows_a, acc.at[didx_a], add=True)
        gnext = jnp.minimum(g + 2, coff + cnt - 1)
        pltpu.sync_copy(src_h.at[sid, gnext], sidx_a)
        pltpu.sync_copy(dst_h.at[sid, gnext], didx_a)
        pltpu.async_copy(tab_h.at[sidx_a], rows_a, sem_a)
        pltpu.make_async_copy(tab_h.at[sidx_b], rows_b, sem_b).wait()
        pltpu.sync_copy(rows_b, acc.at[didx_b], add=True)
        return carry

    lax.fori_loop(0, cnt // 2, body, 0)
    pltpu.make_async_copy(tab_h.at[sidx_a], rows_a, sem_a).wait()
    plsc.subcore_barrier()
    for jz in range(SLAB // CHUNK):
        pltpu.sync_copy(acc.at[pl.ds(base + jz * CHUNK, CHUNK)], rows_a)
        pltpu.sync_copy(rows_a,
                        out_h.at[cid, pl.ds(base + jz * CHUNK, CHUNK)])


_scat_kernel = pl.kernel(
    _scat_pipe_body,
    out_type=jax.ShapeDtypeStruct((NC, NPAD, F), BT),
    mesh=_MESH,
    compiler_params=_SC_PARAMS,
    scratch_types=[
        pltpu.VMEM((CHUNK,), jnp.int32),
        pltpu.VMEM((CHUNK,), jnp.int32),
        pltpu.VMEM((CHUNK,), jnp.int32),
        pltpu.VMEM((CHUNK,), jnp.int32),
        pltpu.VMEM((CHUNK, F), BT),
        pltpu.VMEM((CHUNK, F), BT),
        pltpu.SemaphoreType.DMA,
        pltpu.SemaphoreType.DMA,
        pltpu.VMEM_SHARED((NPAD, F), BT),
    ],
)

_deg_kernel = pl.kernel(
    _deg_body,
    out_type=jax.ShapeDtypeStruct((NC, NPAD, F), BT),
    mesh=_MESH,
    compiler_params=_SC_PARAMS,
    scratch_types=[
        pltpu.VMEM((CHUNK,), jnp.int32),
        pltpu.VMEM((CHUNK, F), BT),
        pltpu.VMEM_SHARED((NPAD, F), BT),
    ],
)


def _prep_body(degp_ref, x_ref, xs_ref, dinv_ref):
    deg = (degp_ref[0][:, :1].astype(jnp.float32)
           + degp_ref[1][:, :1].astype(jnp.float32))
    dinv = lax.rsqrt(deg + 1.0)
    dinv_ref[...] = dinv
    xs_ref[:N, :] = (x_ref[...] * dinv[:N]).astype(BT)
    xs_ref[N:, :] = jnp.zeros((TPAD - N, F), BT)


_prep_kernel = pl.pallas_call(
    _prep_body,
    out_shape=(jax.ShapeDtypeStruct((TPAD, F), BT),
               jax.ShapeDtypeStruct((NPAD, 1), jnp.float32)),
)


def _mid_body(acc_ref, xs_ref, dinv_ref, w1_ref, b1_ref, g_ref, bt_ref,
              w2_ref, ys_ref):
    dinv = dinv_ref[...][:N]
    a = (acc_ref[0][:N].astype(jnp.float32) + acc_ref[1][:N].astype(jnp.float32)
         + xs_ref[...][:N].astype(jnp.float32))
    p1 = a * dinv
    t = jnp.dot(p1, w1_ref[...], preferred_element_type=jnp.float32,
                precision=lax.Precision.HIGHEST) + b1_ref[...]
    mean = jnp.mean(t, axis=0, keepdims=True)
    c = t - mean
    var = jnp.mean(c * c, axis=0, keepdims=True)
    h = jnp.maximum(c * lax.rsqrt(var + 1e-5) * g_ref[...] + bt_ref[...], 0.0)
    q = jnp.dot(h, w2_ref[...], preferred_element_type=jnp.float32,
                precision=lax.Precision.HIGHEST)
    ys_ref[:N, :] = (q * dinv).astype(BT)
    ys_ref[N:, :] = jnp.zeros((TPAD - N, F), BT)


_mid_kernel = pl.pallas_call(
    _mid_body,
    out_shape=jax.ShapeDtypeStruct((TPAD, F), BT),
)


def _fin_body(acc_ref, ys_ref, dinv_ref, b2_ref, out_ref):
    a = (acc_ref[0][:N].astype(jnp.float32) + acc_ref[1][:N].astype(jnp.float32)
         + ys_ref[...][:N].astype(jnp.float32))
    out_ref[...] = a * dinv_ref[...][:N] + b2_ref[...]


_fin_kernel = pl.pallas_call(
    _fin_body,
    out_shape=jax.ShapeDtypeStruct((N, F), jnp.float32),
)


def kernel(x, edge_index, W1, b1, gamma, beta, W2, b2):
    ei = edge_index.astype(jnp.int32)
    pad = jnp.full((EPAD - E,), N, jnp.int32)
    srcp = jnp.concatenate([ei[0], pad]).reshape(NS, CPT, CHUNK)
    dstp = jnp.concatenate([ei[1], pad]).reshape(NS, CPT, CHUNK)
    ones_row = jnp.ones((CHUNK, F), BT)
    zeros_row = jnp.zeros((CHUNK, F), BT)

    degp = _deg_kernel(ones_row, srcp, dstp, zeros_row)
    xs, dinv = _prep_kernel(degp, x)
    acc1 = _scat_kernel(xs, srcp, dstp, zeros_row)
    ys = _mid_kernel(acc1, xs, dinv, W1, b1.reshape(1, -1),
                     gamma.reshape(1, -1), beta.reshape(1, -1), W2)
    acc2 = _scat_kernel(ys, srcp, dstp, zeros_row)
    return _fin_kernel(acc2, ys, dinv, b2.reshape(1, -1))


# speedup vs baseline: 1.6576x; 1.1150x over previous
import jax
import jax.numpy as jnp
from jax import lax
from jax.experimental import pallas as pl
from jax.experimental.pallas import tpu as pltpu
from jax.experimental.pallas import tpu_sc as plsc

N = 10000
F = 128
E = 320000
NC, NS = 2, 16
NW = NC * NS
CHUNK = 128
CPW = 80
CPT = 2 * CPW
CP_FAST, CP_SLOW = 108, 52
FAST_CID = 1
BT = jnp.bfloat16
_SC_PARAMS = pltpu.CompilerParams(use_tc_tiling_on_sc=False)
EPAD = NW * CPW * CHUNK
NPAD = 10240
SLAB = NPAD // NS
TPAD = 10016

_MESH = plsc.VectorSubcoreMesh(
    core_axis_name="c", subcore_axis_name="s", num_cores=NC, num_subcores=NS)


def _deg_body(ones_h, src_h, dst_h, zrow_h, out_h, didx_v, rows_v, acc):
    cid = lax.axis_index("c")
    sid = lax.axis_index("s")
    base = sid * SLAB
    coff = cid * CPW
    pltpu.sync_copy(zrow_h, rows_v)
    for jz in range(SLAB // CHUNK):
        pltpu.sync_copy(rows_v, acc.at[pl.ds(base + jz * CHUNK, CHUNK)])
    plsc.subcore_barrier()
    pltpu.sync_copy(ones_h.at[pl.ds(0, CHUNK)], rows_v)

    def body(j, carry):
        pltpu.sync_copy(dst_h.at[sid, coff + j], didx_v)
        pltpu.sync_copy(rows_v, acc.at[didx_v], add=True)
        return carry

    lax.fori_loop(0, CPW, body, 0)
    plsc.subcore_barrier()
    for jz in range(SLAB // CHUNK):
        pltpu.sync_copy(acc.at[pl.ds(base + jz * CHUNK, CHUNK)], rows_v)
        pltpu.sync_copy(rows_v,
                        out_h.at[cid, pl.ds(base + jz * CHUNK, CHUNK)])


def _scat_pipe_body(tab_h, src_h, dst_h, zrow_h, out_h,
                    sidx_a, didx_a, sidx_b, didx_b, rows_a, rows_b,
                    sem_a, sem_b, acc):
    cid = lax.axis_index("c")
    sid = lax.axis_index("s")
    base = sid * SLAB
    fast = cid == FAST_CID
    coff = jnp.where(fast, 0, CP_FAST)
    cnt = jnp.where(fast, CP_FAST, CP_SLOW)
    pltpu.sync_copy(zrow_h, rows_a)
    for jz in range(SLAB // CHUNK):
        pltpu.sync_copy(rows_a, acc.at[pl.ds(base + jz * CHUNK, CHUNK)])
    plsc.subcore_barrier()
    pltpu.sync_copy(src_h.at[sid, coff], sidx_a)
    pltpu.sync_copy(dst_h.at[sid, coff], didx_a)
    pltpu.async_copy(tab_h.at[sidx_a], rows_a, sem_a)

    def body(g2, carry):
        g = coff + 2 * g2
        pltpu.sync_copy(src_h.at[sid, g + 1], sidx_b)
        pltpu.sync_copy(dst_h.at[sid, g + 1], didx_b)
        pltpu.async_copy(tab_h.at[sidx_b], rows_b, sem_b)
        pltpu.make_async_copy(tab_h.at[sidx_a], rows_a, sem_a).wait()
        pltpu.sync_copy(rows_a, acc.at[didx_a], add=True)
        gnext = jnp.minimum(g + 2, coff + cnt - 1)
        pltpu.sync_copy(src_h.at[sid, gnext], sidx_a)
        pltpu.sync_copy(dst_h.at[sid, gnext], didx_a)
        pltpu.async_copy(tab_h.at[sidx_a], rows_a, sem_a)
        pltpu.make_async_copy(tab_h.at[sidx_b], rows_b, sem_b).wait()
        pltpu.sync_copy(rows_b, acc.at[didx_b], add=True)
        return carry

    lax.fori_loop(0, cnt // 2, body, 0)
    pltpu.make_async_copy(tab_h.at[sidx_a], rows_a, sem_a).wait()
    plsc.subcore_barrier()
    for jz in range(SLAB // CHUNK):
        pltpu.sync_copy(acc.at[pl.ds(base + jz * CHUNK, CHUNK)], rows_a)
        pltpu.sync_copy(rows_a,
                        out_h.at[cid, pl.ds(base + jz * CHUNK, CHUNK)])


_scat_kernel = pl.kernel(
    _scat_pipe_body,
    out_type=jax.ShapeDtypeStruct((NC, NPAD, F), BT),
    mesh=_MESH,
    compiler_params=_SC_PARAMS,
    scratch_types=[
        pltpu.VMEM((CHUNK,), jnp.int32),
        pltpu.VMEM((CHUNK,), jnp.int32),
        pltpu.VMEM((CHUNK,), jnp.int32),
        pltpu.VMEM((CHUNK,), jnp.int32),
        pltpu.VMEM((CHUNK, F), BT),
        pltpu.VMEM((CHUNK, F), BT),
        pltpu.SemaphoreType.DMA,
        pltpu.SemaphoreType.DMA,
        pltpu.VMEM_SHARED((NPAD, F), BT),
    ],
)

_deg_kernel = pl.kernel(
    _deg_body,
    out_type=jax.ShapeDtypeStruct((NC, NPAD, F), BT),
    mesh=_MESH,
    compiler_params=_SC_PARAMS,
    scratch_types=[
        pltpu.VMEM((CHUNK,), jnp.int32),
        pltpu.VMEM((CHUNK, F), BT),
        pltpu.VMEM_SHARED((NPAD, F), BT),
    ],
)


def _prep_body(degp_ref, x_ref, xs_ref, dinv_ref):
    deg = (degp_ref[0][:, :1].astype(jnp.float32)
           + degp_ref[1][:, :1].astype(jnp.float32))
    dinv = lax.rsqrt(deg + 1.0)
    dinv_ref[...] = dinv
    xs_ref[:N, :] = (x_ref[...] * dinv[:N]).astype(BT)
    xs_ref[N:, :] = jnp.zeros((TPAD - N, F), BT)


_prep_kernel = pl.pallas_call(
    _prep_body,
    out_shape=(jax.ShapeDtypeStruct((TPAD, F), BT),
               jax.ShapeDtypeStruct((NPAD, 1), jnp.float32)),
)


def _mid_body(acc_ref, xs_ref, dinv_ref, w1_ref, b1_ref, g_ref, bt_ref,
              w2_ref, ys_ref):
    dinv = dinv_ref[...][:N]
    a = (acc_ref[0][:N].astype(jnp.float32) + acc_ref[1][:N].astype(jnp.float32)
         + xs_ref[...][:N].astype(jnp.float32))
    p1 = a * dinv
    t = jnp.dot(p1, w1_ref[...], preferred_element_type=jnp.float32,
                precision=lax.Precision.HIGHEST) + b1_ref[...]
    mean = jnp.mean(t, axis=0, keepdims=True)
    c = t - mean
    var = jnp.mean(c * c, axis=0, keepdims=True)
    h = jnp.maximum(c * lax.rsqrt(var + 1e-5) * g_ref[...] + bt_ref[...], 0.0)
    q = jnp.dot(h, w2_ref[...], preferred_element_type=jnp.float32,
                precision=lax.Precision.HIGHEST)
    ys_ref[:N, :] = (q * dinv).astype(BT)
    ys_ref[N:, :] = jnp.zeros((TPAD - N, F), BT)


_mid_kernel = pl.pallas_call(
    _mid_body,
    out_shape=jax.ShapeDtypeStruct((TPAD, F), BT),
)


def _fin_body(acc_ref, ys_ref, dinv_ref, b2_ref, out_ref):
    a = (acc_ref[0][:N].astype(jnp.float32) + acc_ref[1][:N].astype(jnp.float32)
         + ys_ref[...][:N].astype(jnp.float32))
    out_ref[...] = a * dinv_ref[...][:N] + b2_ref[...]


_fin_kernel = pl.pallas_call(
    _fin_body,
    out_shape=jax.ShapeDtypeStruct((N, F), jnp.float32),
)


def kernel(x, edge_index, W1, b1, gamma, beta, W2, b2):
    ei = edge_index.astype(jnp.int32)
    pad = jnp.full((EPAD - E,), N, jnp.int32)
    srcp = jnp.concatenate([ei[0], pad]).reshape(NS, CPT, CHUNK)
    dstp = jnp.concatenate([ei[1], pad]).reshape(NS, CPT, CHUNK)
    ones_row = jnp.ones((CHUNK, F), BT)
    zeros_row = jnp.zeros((CHUNK, F), BT)

    degp = _deg_kernel(ones_row, srcp, dstp, zeros_row)
    xs, dinv = _prep_kernel(degp, x)
    acc1 = _scat_kernel(xs, srcp, dstp, zeros_row)
    ys = _mid_kernel(acc1, xs, dinv, W1, b1.reshape(1, -1),
                     gamma.reshape(1, -1), beta.reshape(1, -1), W2)
    acc2 = _scat_kernel(ys, srcp, dstp, zeros_row)
    return _fin_kernel(acc2, ys, dinv, b2.reshape(1, -1))


# speedup vs baseline: 1.7198x; 1.0375x over previous
import jax
import jax.numpy as jnp
from jax import lax
from jax.experimental import pallas as pl
from jax.experimental.pallas import tpu as pltpu
from jax.experimental.pallas import tpu_sc as plsc

N = 10000
F = 128
E = 320000
NC, NS = 2, 16
NW = NC * NS
CHUNK = 128
CPW = 80
CPT = 2 * CPW
CP_FAST, CP_SLOW = 108, 52
FAST_CID = 1
BT = jnp.bfloat16
_SC_PARAMS = pltpu.CompilerParams(use_tc_tiling_on_sc=False)
EPAD = NW * CPW * CHUNK
NPAD = 10240
SLAB = NPAD // NS
TPAD = 10016

_MESH = plsc.VectorSubcoreMesh(
    core_axis_name="c", subcore_axis_name="s", num_cores=NC, num_subcores=NS)


def _deg_body(ones_h, src_h, dst_h, zrow_h, out_h, didx_a, didx_b,
              sem_a, sem_b, rows_v, acc):
    cid = lax.axis_index("c")
    sid = lax.axis_index("s")
    base = sid * SLAB
    coff = cid * CPW
    pltpu.sync_copy(zrow_h, rows_v)
    for jz in range(SLAB // CHUNK):
        pltpu.sync_copy(rows_v, acc.at[pl.ds(base + jz * CHUNK, CHUNK)])
    plsc.subcore_barrier()
    pltpu.sync_copy(ones_h.at[pl.ds(0, CHUNK)], rows_v)
    pltpu.sync_copy(dst_h.at[sid, coff], didx_a)
    pltpu.async_copy(rows_v, acc.at[didx_a], sem_a, add=True)

    def body(g2, carry):
        g = coff + 2 * g2
        pltpu.sync_copy(dst_h.at[sid, g + 1], didx_b)
        pltpu.async_copy(rows_v, acc.at[didx_b], sem_b, add=True)
        pltpu.make_async_copy(rows_v, acc.at[didx_a], sem_a).wait()

        @pl.when(g + 2 < coff + CPW)
        def _():
            pltpu.sync_copy(dst_h.at[sid, g + 2], didx_a)
            pltpu.async_copy(rows_v, acc.at[didx_a], sem_a, add=True)

        pltpu.make_async_copy(rows_v, acc.at[didx_b], sem_b).wait()
        return carry

    lax.fori_loop(0, CPW // 2, body, 0)
    plsc.subcore_barrier()
    for jz in range(SLAB // CHUNK):
        pltpu.sync_copy(acc.at[pl.ds(base + jz * CHUNK, CHUNK)], rows_v)
        pltpu.sync_copy(rows_v,
                        out_h.at[cid, pl.ds(base + jz * CHUNK, CHUNK)])


def _scat_pipe_body(tab_h, src_h, dst_h, zrow_h, out_h,
                    sidx_a, didx_a, sidx_b, didx_b, rows_a, rows_b,
                    sem_a, sem_b, acc):
    cid = lax.axis_index("c")
    sid = lax.axis_index("s")
    base = sid * SLAB
    fast = cid == FAST_CID
    coff = jnp.where(fast, 0, CP_FAST)
    cnt = jnp.where(fast, CP_FAST, CP_SLOW)
    pltpu.sync_copy(zrow_h, rows_a)
    for jz in range(SLAB // CHUNK):
        pltpu.sync_copy(rows_a, acc.at[pl.ds(base + jz * CHUNK, CHUNK)])
    plsc.subcore_barrier()
    pltpu.sync_copy(src_h.at[sid, coff], sidx_a)
    pltpu.sync_copy(dst_h.at[sid, coff], didx_a)
    pltpu.async_copy(tab_h.at[sidx_a], rows_a, sem_a)

    def body(g2, carry):
        g = coff + 2 * g2
        pltpu.sync_copy(src_h.at[sid, g + 1], sidx_b)
        pltpu.sync_copy(dst_h.at[sid, g + 1], didx_b)
        pltpu.async_copy(tab_h.at[sidx_b], rows_b, sem_b)
        pltpu.make_async_copy(tab_h.at[sidx_a], rows_a, sem_a).wait()
        pltpu.sync_copy(rows_a, acc.at[didx_a], add=True)
        gnext = jnp.minimum(g + 2, coff + cnt - 1)
        pltpu.sync_copy(src_h.at[sid, gnext], sidx_a)
        pltpu.sync_copy(dst_h.at[sid, gnext], didx_a)
        pltpu.async_copy(tab_h.at[sidx_a], rows_a, sem_a)
        pltpu.make_async_copy(tab_h.at[sidx_b], rows_b, sem_b).wait()
        pltpu.sync_copy(rows_b, acc.at[didx_b], add=True)
        return carry

    lax.fori_loop(0, cnt // 2, body, 0)
    pltpu.make_async_copy(tab_h.at[sidx_a], rows_a, sem_a).wait()
    plsc.subcore_barrier()
    for jz in range(SLAB // CHUNK):
        pltpu.sync_copy(acc.at[pl.ds(base + jz * CHUNK, CHUNK)], rows_a)
        pltpu.sync_copy(rows_a,
                        out_h.at[cid, pl.ds(base + jz * CHUNK, CHUNK)])


_scat_kernel = pl.kernel(
    _scat_pipe_body,
    out_type=jax.ShapeDtypeStruct((NC, NPAD, F), BT),
    mesh=_MESH,
    compiler_params=_SC_PARAMS,
    scratch_types=[
        pltpu.VMEM((CHUNK,), jnp.int32),
        pltpu.VMEM((CHUNK,), jnp.int32),
        pltpu.VMEM((CHUNK,), jnp.int32),
        pltpu.VMEM((CHUNK,), jnp.int32),
        pltpu.VMEM((CHUNK, F), BT),
        pltpu.VMEM((CHUNK, F), BT),
        pltpu.SemaphoreType.DMA,
        pltpu.SemaphoreType.DMA,
        pltpu.VMEM_SHARED((NPAD, F), BT),
    ],
)

_deg_kernel = pl.kernel(
    _deg_body,
    out_type=jax.ShapeDtypeStruct((NC, NPAD, F), BT),
    mesh=_MESH,
    compiler_params=_SC_PARAMS,
    scratch_types=[
        pltpu.VMEM((CHUNK,), jnp.int32),
        pltpu.VMEM((CHUNK,), jnp.int32),
        pltpu.SemaphoreType.DMA,
        pltpu.SemaphoreType.DMA,
        pltpu.VMEM((CHUNK, F), BT),
        pltpu.VMEM_SHARED((NPAD, F), BT),
    ],
)


def _prep_body(degp_ref, x_ref, xs_ref, dinv_ref):
    deg = (degp_ref[0][:, :1].astype(jnp.float32)
           + degp_ref[1][:, :1].astype(jnp.float32))
    dinv = lax.rsqrt(deg + 1.0)
    dinv_ref[...] = dinv
    xs_ref[:N, :] = (x_ref[...] * dinv[:N]).astype(BT)
    xs_ref[N:, :] = jnp.zeros((TPAD - N, F), BT)


_prep_kernel = pl.pallas_call(
    _prep_body,
    out_shape=(jax.ShapeDtypeStruct((TPAD, F), BT),
               jax.ShapeDtypeStruct((NPAD, 1), jnp.float32)),
)


def _mid_body(acc_ref, xs_ref, dinv_ref, w1_ref, b1_ref, g_ref, bt_ref,
              w2_ref, ys_ref):
    dinv = dinv_ref[...][:N]
    a = (acc_ref[0][:N].astype(jnp.float32) + acc_ref[1][:N].astype(jnp.float32)
         + xs_ref[...][:N].astype(jnp.float32))
    p1 = a * dinv
    t = jnp.dot(p1, w1_ref[...], preferred_element_type=jnp.float32,
                precision=lax.Precision.HIGHEST) + b1_ref[...]
    mean = jnp.mean(t, axis=0, keepdims=True)
    c = t - mean
    var = jnp.mean(c * c, axis=0, keepdims=True)
    h = jnp.maximum(c * lax.rsqrt(var + 1e-5) * g_ref[...] + bt_ref[...], 0.0)
    q = jnp.dot(h, w2_ref[...], preferred_element_type=jnp.float32,
                precision=lax.Precision.HIGHEST)
    ys_ref[:N, :] = (q * dinv).astype(BT)
    ys_ref[N:, :] = jnp.zeros((TPAD - N, F), BT)


_mid_kernel = pl.pallas_call(
    _mid_body,
    out_shape=jax.ShapeDtypeStruct((TPAD, F), BT),
)


def _fin_body(acc_ref, ys_ref, dinv_ref, b2_ref, out_ref):
    a = (acc_ref[0][:N].astype(jnp.float32) + acc_ref[1][:N].astype(jnp.float32)
         + ys_ref[...][:N].astype(jnp.float32))
    out_ref[...] = a * dinv_ref[...][:N] + b2_ref[...]


_fin_kernel = pl.pallas_call(
    _fin_body,
    out_shape=jax.ShapeDtypeStruct((N, F), jnp.float32),
)


def kernel(x, edge_index, W1, b1, gamma, beta, W2, b2):
    ei = edge_index.astype(jnp.int32)
    pad = jnp.full((EPAD - E,), N, jnp.int32)
    srcp = jnp.concatenate([ei[0], pad]).reshape(NS, CPT, CHUNK)
    dstp = jnp.concatenate([ei[1], pad]).reshape(NS, CPT, CHUNK)
    ones_row = jnp.ones((CHUNK, F), BT)
    zeros_row = jnp.zeros((CHUNK, F), BT)

    degp = _deg_kernel(ones_row, srcp, dstp, zeros_row)
    xs, dinv = _prep_kernel(degp, x)
    acc1 = _scat_kernel(xs, srcp, dstp, zeros_row)
    ys = _mid_kernel(acc1, xs, dinv, W1, b1.reshape(1, -1),
                     gamma.reshape(1, -1), beta.reshape(1, -1), W2)
    acc2 = _scat_kernel(ys, srcp, dstp, zeros_row)
    return _fin_kernel(acc2, ys, dinv, b2.reshape(1, -1))
